# baseline (speedup 1.0000x reference)
import functools

import jax
import jax.numpy as jnp
from jax import lax
from jax.experimental import pallas as pl
from jax.experimental.pallas import tpu as pltpu
from jax.experimental.pallas import tpu_sc as plsc

N = 10000
E = 160000
C = 256
H = 128
NC = 2
NS = 16
NW = NC * NS
N_PAD = 10240
E_PAD = 163840
CH = 128
EA = E_PAD // NW
NCH_A = EA // CH
EB = E_PAD // NS
NCH_B = EB // CH
RPT = N_PAD // NS
MU = 0.1
P_EXP = 2.5
EPS = 1e-12
CP = 2.0 * MU / P_EXP
K_ITERS = 2
BT = 512
GRID_T = N_PAD // BT

def _sc_params():
  import dataclasses
  cp = pltpu.CompilerParams()
  if "needs_layout_passes" in pltpu.CompilerParams.__dataclass_fields__:
    cp = dataclasses.replace(cp, needs_layout_passes=False)
  return cp


def _mesh():
  return plsc.VectorSubcoreMesh(core_axis_name="c", subcore_axis_name="s",
                                num_cores=NC, num_subcores=NS)


def _sqrt16(u):
  i = plsc.bitcast(u, jnp.int32)
  i = jnp.int32(0x1FBD1DF5) + lax.shift_right_logical(i, 1)
  y = plsc.bitcast(i, jnp.float32)
  for _ in range(3):
    y = 0.5 * (y + u / y)
  return y


def _zero_ref(ref, nwords):
  z16 = jnp.zeros((16,), jnp.float32)

  @pl.loop(0, nwords // 16)
  def _(i):
    ref[pl.ds(i * 16, 16)] = z16


def _deg_body(row_hbm, deg_hbm, degb, rowb, sem):
  del sem
  wid = lax.axis_index("s") * NC + lax.axis_index("c")
  _zero_ref(degb, N_PAD)
  one16 = jnp.ones((16,), jnp.float32)

  @pl.loop(0, NCH_A)
  def _(k):
    base = wid * EA + k * CH
    pltpu.sync_copy(row_hbm.at[pl.ds(base, CH)], rowb)
    for g in range(CH // 16):
      ridx = rowb[pl.ds(g * 16, 16)]
      plsc.addupdate_scatter(degb, [ridx], one16)

  pltpu.sync_copy(degb, deg_hbm.at[wid])


def _passA_body(f0, f1, row_hbm, col_hbm, dis_hbm, c_hbm, msum_hbm,
                disb, msumb, rowb, colb, fr0, fr1, fc0, fc1, cbuf, sem):
  wid = lax.axis_index("s") * NC + lax.axis_index("c")
  _zero_ref(msumb, N_PAD)
  pltpu.sync_copy(dis_hbm, disb)
  iota16 = lax.iota(jnp.int32, 16)
  zero16 = jnp.zeros((16,), jnp.float32)

  @pl.loop(0, NCH_A)
  def _(k):
    base = wid * EA + k * CH
    pltpu.sync_copy(row_hbm.at[pl.ds(base, CH)], rowb)
    pltpu.sync_copy(col_hbm.at[pl.ds(base, CH)], colb)
    c1 = pltpu.async_copy(f0.at[rowb], fr0, sem)
    c2 = pltpu.async_copy(f1.at[rowb], fr1, sem)
    c3 = pltpu.async_copy(f0.at[colb], fc0, sem)
    c4 = pltpu.async_copy(f1.at[colb], fc1, sem)
    c1.wait()
    c2.wait()
    c3.wait()
    c4.wait()
    for g in range(CH // 16):
      erow = iota16 + g * 16
      ridx = rowb[pl.ds(g * 16, 16)]
      cidx = colb[pl.ds(g * 16, 16)]
      a = plsc.load_gather(disb, [ridx])
      b = plsc.load_gather(disb, [cidx])

      def chan(ci, sg, a=a, b=b, erow=erow):
        cvec = jnp.broadcast_to(ci, (16,))
        vr0 = plsc.load_gather(fr0, [erow, cvec])
        vc0 = plsc.load_gather(fc0, [erow, cvec])
        vr1 = plsc.load_gather(fr1, [erow, cvec])
        vc1 = plsc.load_gather(fc1, [erow, cvec])
        g0 = b * vc0 - a * vr0
        g1 = b * vc1 - a * vr1
        return sg + g0 * g0 + g1 * g1

      sg = lax.fori_loop(0, H, chan, zero16)
      u = sg + EPS
      m = _sqrt16(_sqrt16(u))
      cbuf[pl.ds(g * 16, 16)] = m * a * b
      plsc.addupdate_scatter(msumb, [ridx], m)
    pltpu.sync_copy(cbuf, c_hbm.at[pl.ds(base, CH)])

  pltpu.sync_copy(msumb, msum_hbm.at[wid])


def _passB_body(f0, f1, row_hbm, col_hbm, c_hbm, agg0_hbm, agg1_hbm,
                aggsh, fcb, rowb, colb, cb, zb, sem):
  cid = lax.axis_index("c")
  sid = lax.axis_index("s")
  z16 = jnp.zeros((16,), jnp.float32)
  for j in range(64):
    for l in range(H // 16):
      zb[j, pl.ds(l * 16, 16)] = z16

  @pl.loop(0, RPT // 64)
  def _(j):
    pltpu.sync_copy(zb, aggsh.at[pl.ds(sid * RPT + j * 64, 64)])

  plsc.subcore_barrier()
  iota16 = lax.iota(jnp.int32, 16)

  def half(f_hbm, agg_hbm):
    @pl.loop(0, NCH_B)
    def _(k):
      base = sid * EB + k * CH
      pltpu.sync_copy(row_hbm.at[pl.ds(base, CH)], rowb)
      pltpu.sync_copy(col_hbm.at[pl.ds(base, CH)], colb)
      pltpu.sync_copy(c_hbm.at[pl.ds(base, CH)], cb)
      pltpu.async_copy(f_hbm.at[colb], fcb, sem).wait()
      for g in range(CH // 16):
        erow = iota16 + g * 16
        ce = cb[pl.ds(g * 16, 16)]

        def chan(ci, carry, erow=erow, ce=ce):
          cvec = jnp.broadcast_to(ci, (16,))
          v = plsc.load_gather(fcb, [erow, cvec])
          plsc.store_scatter(fcb, [erow, cvec], v * ce)
          return carry

        lax.fori_loop(0, H, chan, 0)
      pltpu.sync_copy(fcb, aggsh.at[rowb], add=True)

    plsc.subcore_barrier()

    @pl.loop(0, RPT // 64)
    def _(j):
      r = sid * RPT + j * 64
      pltpu.sync_copy(aggsh.at[pl.ds(r, 64)], agg_hbm.at[pl.ds(r, 64)])

  @pl.when(cid == 0)
  def _():
    half(f0, agg0_hbm)

  @pl.when(cid == 1)
  def _():
    half(f1, agg1_hbm)


_DN_ROW = (((1,), (0,)), ((), ()))
_DN_COL = (((0,), (0,)), ((), ()))


def _tc1_kernel(x_ref, w1_ref, b1_ref, degp_ref, h0_ref, h1_ref, dis_ref):
  h = lax.dot_general(x_ref[...], w1_ref[...], _DN_ROW,
                      preferred_element_type=jnp.float32,
                      precision=lax.Precision.HIGHEST)
  h = jnp.maximum(h + b1_ref[...], 0.0)
  h0_ref[...] = h[:, :H]
  h1_ref[...] = h[:, H:]
  deg = jnp.sum(degp_ref[...], axis=0, keepdims=True)
  dis_ref[...] = jnp.where(deg > 0, lax.rsqrt(jnp.maximum(deg, EPS)), 0.0)


def _tcC_kernel(agg0_ref, agg1_ref, h0_ref, h1_ref, msump_ref, degp_ref,
                f0_ref, f1_ref):
  ones = jnp.ones((NW, 1), jnp.float32)
  msum = lax.dot_general(msump_ref[...], ones, _DN_COL,
                         preferred_element_type=jnp.float32,
                         precision=lax.Precision.HIGHEST)
  deg = lax.dot_general(degp_ref[...], ones, _DN_COL,
                        preferred_element_type=jnp.float32,
                        precision=lax.Precision.HIGHEST)
  alpha = 1.0 / (msum / jnp.maximum(deg, EPS) + CP)
  beta = CP * alpha
  f0_ref[...] = alpha * agg0_ref[...] + beta * h0_ref[...]
  f1_ref[...] = alpha * agg1_ref[...] + beta * h1_ref[...]


def _tc2_kernel(f0_ref, f1_ref, w2_ref, b2_ref, o_ref):
  w2 = w2_ref[...]
  z = lax.dot_general(f0_ref[...], w2[:H, :], _DN_ROW,
                      preferred_element_type=jnp.float32,
                      precision=lax.Precision.HIGHEST)
  z += lax.dot_general(f1_ref[...], w2[H:, :], _DN_ROW,
                       preferred_element_type=jnp.float32,
                       precision=lax.Precision.HIGHEST)
  z += b2_ref[...]
  m = jnp.max(z, axis=1, keepdims=True)
  o_ref[...] = z - m - jnp.log(jnp.sum(jnp.exp(z - m), axis=1, keepdims=True))


_f32 = jnp.float32


def _sc_deg(row_p):
  fn = pl.kernel(
      _deg_body,
      out_type=jax.ShapeDtypeStruct((NW, N_PAD), _f32),
      mesh=_mesh(),
      compiler_params=_sc_params(),
      scratch_types=[
          pltpu.VMEM((N_PAD,), _f32),
          pltpu.VMEM((CH,), jnp.int32),
          pltpu.SemaphoreType.DMA,
      ],
  )
  return fn(row_p)


def _sc_passA(f0, f1, row_p, col_p, dis):
  fn = pl.kernel(
      _passA_body,
      out_type=(jax.ShapeDtypeStruct((E_PAD,), _f32),
                jax.ShapeDtypeStruct((NW, N_PAD), _f32)),
      mesh=_mesh(),
      compiler_params=_sc_params(),
      scratch_types=[
          pltpu.VMEM((N_PAD,), _f32),
          pltpu.VMEM((N_PAD,), _f32),
          pltpu.VMEM((CH,), jnp.int32),
          pltpu.VMEM((CH,), jnp.int32),
          pltpu.VMEM((CH, H), _f32),
          pltpu.VMEM((CH, H), _f32),
          pltpu.VMEM((CH, H), _f32),
          pltpu.VMEM((CH, H), _f32),
          pltpu.VMEM((CH,), _f32),
          pltpu.SemaphoreType.DMA,
      ],
  )
  return fn(f0, f1, row_p, col_p, dis)


def _sc_passB(f0, f1, row_p, col_p, cvals):
  fn = pl.kernel(
      _passB_body,
      out_type=(jax.ShapeDtypeStruct((N_PAD, H), _f32),
                jax.ShapeDtypeStruct((N_PAD, H), _f32)),
      mesh=_mesh(),
      compiler_params=_sc_params(),
      scratch_types=[
          pltpu.VMEM_SHARED((N_PAD, H), _f32),
          pltpu.VMEM((CH, H), _f32),
          pltpu.VMEM((CH,), jnp.int32),
          pltpu.VMEM((CH,), jnp.int32),
          pltpu.VMEM((CH,), _f32),
          pltpu.VMEM((64, H), _f32),
          pltpu.SemaphoreType.DMA,
      ],
  )
  return fn(f0, f1, row_p, col_p, cvals)


def _tc1(x_p, w1, b1, deg_parts):
  return pl.pallas_call(
      _tc1_kernel,
      grid=(GRID_T,),
      in_specs=[
          pl.BlockSpec((BT, C), lambda i: (i, 0)),
          pl.BlockSpec((C, C), lambda i: (0, 0)),
          pl.BlockSpec((1, C), lambda i: (0, 0)),
          pl.BlockSpec((NW, BT), lambda i: (0, i)),
      ],
      out_specs=[
          pl.BlockSpec((BT, H), lambda i: (i, 0)),
          pl.BlockSpec((BT, H), lambda i: (i, 0)),
          pl.BlockSpec((1, BT), lambda i: (0, i)),
      ],
      out_shape=[
          jax.ShapeDtypeStruct((N_PAD, H), _f32),
          jax.ShapeDtypeStruct((N_PAD, H), _f32),
          jax.ShapeDtypeStruct((1, N_PAD), _f32),
      ],
  )(x_p, w1, b1, deg_parts)


def _tcC(agg0, agg1, h0, h1, msum_parts, deg_parts):
  return pl.pallas_call(
      _tcC_kernel,
      grid=(GRID_T,),
      in_specs=[
          pl.BlockSpec((BT, H), lambda i: (i, 0)),
          pl.BlockSpec((BT, H), lambda i: (i, 0)),
          pl.BlockSpec((BT, H), lambda i: (i, 0)),
          pl.BlockSpec((BT, H), lambda i: (i, 0)),
          pl.BlockSpec((NW, BT), lambda i: (0, i)),
          pl.BlockSpec((NW, BT), lambda i: (0, i)),
      ],
      out_specs=[
          pl.BlockSpec((BT, H), lambda i: (i, 0)),
          pl.BlockSpec((BT, H), lambda i: (i, 0)),
      ],
      out_shape=[
          jax.ShapeDtypeStruct((N_PAD, H), _f32),
          jax.ShapeDtypeStruct((N_PAD, H), _f32),
      ],
  )(agg0, agg1, h0, h1, msum_parts, deg_parts)


def _tc2(f0, f1, w2, b2):
  return pl.pallas_call(
      _tc2_kernel,
      grid=(GRID_T,),
      in_specs=[
          pl.BlockSpec((BT, H), lambda i: (i, 0)),
          pl.BlockSpec((BT, H), lambda i: (i, 0)),
          pl.BlockSpec((C, C), lambda i: (0, 0)),
          pl.BlockSpec((1, C), lambda i: (0, 0)),
      ],
      out_specs=pl.BlockSpec((BT, C), lambda i: (i, 0)),
      out_shape=jax.ShapeDtypeStruct((N_PAD, C), _f32),
  )(f0, f1, w2, b2)


def kernel(x, edge_index, w1, b1, w2, b2):
  row = edge_index[0]
  col = edge_index[1]
  row_p = jnp.concatenate(
      [row, jnp.full((E_PAD - E,), N_PAD - 1, jnp.int32)])
  col_p = jnp.concatenate([col, jnp.zeros((E_PAD - E,), jnp.int32)])
  x_p = jnp.pad(x, ((0, N_PAD - N), (0, 0)))

  deg_parts = _sc_deg(row_p)
  h0, h1, dis2d = _tc1(x_p, w1, b1.reshape(1, C), deg_parts)
  dis = dis2d.reshape(N_PAD)

  f0, f1 = h0, h1
  for _ in range(K_ITERS):
    cvals, msum_parts = _sc_passA(f0, f1, row_p, col_p, dis)
    agg0, agg1 = _sc_passB(f0, f1, row_p, col_p, cvals)
    f0, f1 = _tcC(agg0, agg1, h0, h1, msum_parts, deg_parts)

  out = _tc2(f0, f1, w2, b2.reshape(1, C))
  return out[:N]


# speedup vs baseline: 2.7469x; 2.7469x over previous
import dataclasses

import jax
import jax.numpy as jnp
from jax import lax
from jax.experimental import pallas as pl
from jax.experimental.pallas import tpu as pltpu
from jax.experimental.pallas import tpu_sc as plsc

N = 10000
E = 160000
C = 256
H = 128
NC = 2
NS = 16
NW = NC * NS
N_PAD = 10240
E_PAD = 163840
CH = 128
EA = E_PAD // NW
NCH_A = EA // CH
EB = E_PAD // NS
NCH_B = EB // CH
RPT = N_PAD // NS
MU = 0.1
P_EXP = 2.5
EPS = 1e-12
CP = 2.0 * MU / P_EXP
K_ITERS = 2
BT = 512
GRID_T = N_PAD // BT
BE = 2048
GRID_E = E_PAD // BE


def _sc_params():
  cp = pltpu.CompilerParams()
  if "needs_layout_passes" in pltpu.CompilerParams.__dataclass_fields__:
    cp = dataclasses.replace(cp, needs_layout_passes=False)
  return cp


def _mesh():
  return plsc.VectorSubcoreMesh(core_axis_name="c", subcore_axis_name="s",
                                num_cores=NC, num_subcores=NS)


def _zero_ref(ref, nwords):
  z16 = jnp.zeros((16,), jnp.float32)

  @pl.loop(0, nwords // 16)
  def _(i):
    ref[pl.ds(i * 16, 16)] = z16


def _deg_body(row_hbm, deg_hbm, degb, rowb, sem):
  del sem
  wid = lax.axis_index("s") * NC + lax.axis_index("c")
  _zero_ref(degb, N_PAD)
  one16 = jnp.ones((16,), jnp.float32)

  @pl.loop(0, NCH_A)
  def _(k):
    base = wid * EA + k * CH
    pltpu.sync_copy(row_hbm.at[pl.ds(base, CH)], rowb)
    for g in range(CH // 16):
      ridx = rowb[pl.ds(g * 16, 16)]
      plsc.addupdate_scatter(degb, [ridx], one16)

  pltpu.sync_copy(degb, deg_hbm.at[wid])


def _gath_body(f0, f1, row_hbm, col_hbm, dis_hbm,
               fr0_hbm, fr1_hbm, fc0_hbm, fc1_hbm, disr_hbm, disc_hbm,
               disb, rowb, colb, br0, br1, bc0, bc1, drb, dcb, sem):
  wid = lax.axis_index("s") * NC + lax.axis_index("c")
  pltpu.sync_copy(dis_hbm, disb)

  @pl.loop(0, NCH_A)
  def _(k):
    base = wid * EA + k * CH
    pltpu.sync_copy(row_hbm.at[pl.ds(base, CH)], rowb)
    pltpu.sync_copy(col_hbm.at[pl.ds(base, CH)], colb)
    g1 = pltpu.async_copy(f0.at[rowb], br0, sem)
    g2 = pltpu.async_copy(f1.at[rowb], br1, sem)
    g3 = pltpu.async_copy(f0.at[colb], bc0, sem)
    g4 = pltpu.async_copy(f1.at[colb], bc1, sem)
    for g in range(CH // 16):
      sl = pl.ds(g * 16, 16)
      drb[sl] = plsc.load_gather(disb, [rowb[sl]])
      dcb[sl] = plsc.load_gather(disb, [colb[sl]])
    g1.wait()
    g2.wait()
    g3.wait()
    g4.wait()
    sl = pl.ds(base, CH)
    pltpu.sync_copy(br0, fr0_hbm.at[sl])
    pltpu.sync_copy(br1, fr1_hbm.at[sl])
    pltpu.sync_copy(bc0, fc0_hbm.at[sl])
    pltpu.sync_copy(bc1, fc1_hbm.at[sl])
    pltpu.sync_copy(drb, disr_hbm.at[sl])
    pltpu.sync_copy(dcb, disc_hbm.at[sl])


def _scat_body(msg0, msg1, row_hbm, m_hbm, agg0_hbm, agg1_hbm, msum_hbm,
               aggsh, msgb, rowb, mb, msumb, zb, sem):
  del sem
  cid = lax.axis_index("c")
  sid = lax.axis_index("s")
  z16 = jnp.zeros((16,), jnp.float32)
  for j in range(64):
    for l in range(H // 16):
      zb[j, pl.ds(l * 16, 16)] = z16
  _zero_ref(msumb, N_PAD)

  @pl.loop(0, RPT // 64)
  def _(j):
    pltpu.sync_copy(zb, aggsh.at[pl.ds(sid * RPT + j * 64, 64)])

  plsc.subcore_barrier()

  @pl.loop(0, NCH_B)
  def _(k):
    base = sid * EB + k * CH
    pltpu.sync_copy(row_hbm.at[pl.ds(base, CH)], rowb)

    @pl.when(cid == 0)
    def _():
      pltpu.sync_copy(m_hbm.at[pl.ds(base, CH)], mb)
      for g in range(CH // 16):
        sl = pl.ds(g * 16, 16)
        plsc.addupdate_scatter(msumb, [rowb[sl]], mb[sl])
      pltpu.sync_copy(msg0.at[pl.ds(base, CH)], msgb)

    @pl.when(cid == 1)
    def _():
      pltpu.sync_copy(msg1.at[pl.ds(base, CH)], msgb)

    pltpu.sync_copy(msgb, aggsh.at[rowb], add=True)

  plsc.subcore_barrier()

  @pl.loop(0, RPT // 64)
  def _(j):
    r = sid * RPT + j * 64

    @pl.when(cid == 0)
    def _():
      pltpu.sync_copy(aggsh.at[pl.ds(r, 64)], agg0_hbm.at[pl.ds(r, 64)])

    @pl.when(cid == 1)
    def _():
      pltpu.sync_copy(aggsh.at[pl.ds(r, 64)], agg1_hbm.at[pl.ds(r, 64)])

  @pl.when(cid == 0)
  def _():
    pltpu.sync_copy(msumb, msum_hbm.at[sid])


_DN_ROW = (((1,), (0,)), ((), ()))
_DN_COL = (((0,), (0,)), ((), ()))


def _tc1_kernel(x_ref, w1_ref, b1_ref, degp_ref, h0_ref, h1_ref, dis_ref):
  h = lax.dot_general(x_ref[...], w1_ref[...], _DN_ROW,
                      preferred_element_type=jnp.float32,
                      precision=lax.Precision.HIGHEST)
  h = jnp.maximum(h + b1_ref[...], 0.0)
  h0_ref[...] = h[:, :H]
  h1_ref[...] = h[:, H:]
  deg = jnp.sum(degp_ref[...], axis=0, keepdims=True)
  dis_ref[...] = jnp.where(deg > 0, lax.rsqrt(jnp.maximum(deg, EPS)), 0.0)


def _tcN_kernel(fr0_ref, fr1_ref, fc0_ref, fc1_ref, dr_ref, dc_ref,
                msg0_ref, msg1_ref, m_ref):
  a = dr_ref[...]
  b = dc_ref[...]
  fc0 = fc0_ref[...]
  fc1 = fc1_ref[...]
  g0 = b * fc0 - a * fr0_ref[...]
  g1 = b * fc1 - a * fr1_ref[...]
  u = (jnp.sum(g0 * g0, axis=1, keepdims=True)
       + jnp.sum(g1 * g1, axis=1, keepdims=True) + EPS)
  m = jnp.sqrt(jnp.sqrt(u))
  c = m * a * b
  msg0_ref[...] = c * fc0
  msg1_ref[...] = c * fc1
  m_ref[...] = m


def _tcC_kernel(agg0_ref, agg1_ref, h0_ref, h1_ref, msump_ref, degp_ref,
                f0_ref, f1_ref):
  ones_s = jnp.ones((NS, 1), jnp.float32)
  ones_w = jnp.ones((NW, 1), jnp.float32)
  msum = lax.dot_general(msump_ref[...], ones_s, _DN_COL,
                         preferred_element_type=jnp.float32,
                         precision=lax.Precision.HIGHEST)
  deg = lax.dot_general(degp_ref[...], ones_w, _DN_COL,
                        preferred_element_type=jnp.float32,
                        precision=lax.Precision.HIGHEST)
  alpha = 1.0 / (msum / jnp.maximum(deg, EPS) + CP)
  beta = CP * alpha
  f0_ref[...] = alpha * agg0_ref[...] + beta * h0_ref[...]
  f1_ref[...] = alpha * agg1_ref[...] + beta * h1_ref[...]


def _tc2_kernel(f0_ref, f1_ref, w2_ref, b2_ref, o_ref):
  w2 = w2_ref[...]
  z = lax.dot_general(f0_ref[...], w2[:H, :], _DN_ROW,
                      preferred_element_type=jnp.float32,
                      precision=lax.Precision.HIGHEST)
  z += lax.dot_general(f1_ref[...], w2[H:, :], _DN_ROW,
                       preferred_element_type=jnp.float32,
                       precision=lax.Precision.HIGHEST)
  z += b2_ref[...]
  m = jnp.max(z, axis=1, keepdims=True)
  o_ref[...] = z - m - jnp.log(jnp.sum(jnp.exp(z - m), axis=1, keepdims=True))


_f32 = jnp.float32


def _sc_deg(row_p):
  fn = pl.kernel(
      _deg_body,
      out_type=jax.ShapeDtypeStruct((NW, N_PAD), _f32),
      mesh=_mesh(),
      compiler_params=_sc_params(),
      scratch_types=[
          pltpu.VMEM((N_PAD,), _f32),
          pltpu.VMEM((CH,), jnp.int32),
          pltpu.SemaphoreType.DMA,
      ],
  )
  return fn(row_p)


def _sc_gather(f0, f1, row_p, col_p, dis):
  fn = pl.kernel(
      _gath_body,
      out_type=(jax.ShapeDtypeStruct((E_PAD, H), _f32),
                jax.ShapeDtypeStruct((E_PAD, H), _f32),
                jax.ShapeDtypeStruct((E_PAD, H), _f32),
                jax.ShapeDtypeStruct((E_PAD, H), _f32),
                jax.ShapeDtypeStruct((E_PAD,), _f32),
                jax.ShapeDtypeStruct((E_PAD,), _f32)),
      mesh=_mesh(),
      compiler_params=_sc_params(),
      scratch_types=[
          pltpu.VMEM((N_PAD,), _f32),
          pltpu.VMEM((CH,), jnp.int32),
          pltpu.VMEM((CH,), jnp.int32),
          pltpu.VMEM((CH, H), _f32),
          pltpu.VMEM((CH, H), _f32),
          pltpu.VMEM((CH, H), _f32),
          pltpu.VMEM((CH, H), _f32),
          pltpu.VMEM((CH,), _f32),
          pltpu.VMEM((CH,), _f32),
          pltpu.SemaphoreType.DMA,
      ],
  )
  return fn(f0, f1, row_p, col_p, dis)


def _sc_scatter(msg0, msg1, row_p, mvals):
  fn = pl.kernel(
      _scat_body,
      out_type=(jax.ShapeDtypeStruct((N_PAD, H), _f32),
                jax.ShapeDtypeStruct((N_PAD, H), _f32),
                jax.ShapeDtypeStruct((NS, N_PAD), _f32)),
      mesh=_mesh(),
      compiler_params=_sc_params(),
      scratch_types=[
          pltpu.VMEM_SHARED((N_PAD, H), _f32),
          pltpu.VMEM((CH, H), _f32),
          pltpu.VMEM((CH,), jnp.int32),
          pltpu.VMEM((CH,), _f32),
          pltpu.VMEM((N_PAD,), _f32),
          pltpu.VMEM((64, H), _f32),
          pltpu.SemaphoreType.DMA,
      ],
  )
  return fn(msg0, msg1, row_p, mvals)


def _tc1(x_p, w1, b1, deg_parts):
  return pl.pallas_call(
      _tc1_kernel,
      grid=(GRID_T,),
      in_specs=[
          pl.BlockSpec((BT, C), lambda i: (i, 0)),
          pl.BlockSpec((C, C), lambda i: (0, 0)),
          pl.BlockSpec((1, C), lambda i: (0, 0)),
          pl.BlockSpec((NW, BT), lambda i: (0, i)),
      ],
      out_specs=[
          pl.BlockSpec((BT, H), lambda i: (i, 0)),
          pl.BlockSpec((BT, H), lambda i: (i, 0)),
          pl.BlockSpec((1, BT), lambda i: (0, i)),
      ],
      out_shape=[
          jax.ShapeDtypeStruct((N_PAD, H), _f32),
          jax.ShapeDtypeStruct((N_PAD, H), _f32),
          jax.ShapeDtypeStruct((1, N_PAD), _f32),
      ],
  )(x_p, w1, b1, deg_parts)


def _tcN(fr0, fr1, fc0, fc1, disr, disc):
  dr = disr.reshape(E_PAD, 1)
  dc = disc.reshape(E_PAD, 1)
  msg0, msg1, m2d = pl.pallas_call(
      _tcN_kernel,
      grid=(GRID_E,),
      in_specs=[
          pl.BlockSpec((BE, H), lambda i: (i, 0)),
          pl.BlockSpec((BE, H), lambda i: (i, 0)),
          pl.BlockSpec((BE, H), lambda i: (i, 0)),
          pl.BlockSpec((BE, H), lambda i: (i, 0)),
          pl.BlockSpec((BE, 1), lambda i: (i, 0)),
          pl.BlockSpec((BE, 1), lambda i: (i, 0)),
      ],
      out_specs=[
          pl.BlockSpec((BE, H), lambda i: (i, 0)),
          pl.BlockSpec((BE, H), lambda i: (i, 0)),
          pl.BlockSpec((BE, 1), lambda i: (i, 0)),
      ],
      out_shape=[
          jax.ShapeDtypeStruct((E_PAD, H), _f32),
          jax.ShapeDtypeStruct((E_PAD, H), _f32),
          jax.ShapeDtypeStruct((E_PAD, 1), _f32),
      ],
  )(fr0, fr1, fc0, fc1, dr, dc)
  return msg0, msg1, m2d.reshape(E_PAD)


def _tcC(agg0, agg1, h0, h1, msum_parts, deg_parts):
  return pl.pallas_call(
      _tcC_kernel,
      grid=(GRID_T,),
      in_specs=[
          pl.BlockSpec((BT, H), lambda i: (i, 0)),
          pl.BlockSpec((BT, H), lambda i: (i, 0)),
          pl.BlockSpec((BT, H), lambda i: (i, 0)),
          pl.BlockSpec((BT, H), lambda i: (i, 0)),
          pl.BlockSpec((NS, BT), lambda i: (0, i)),
          pl.BlockSpec((NW, BT), lambda i: (0, i)),
      ],
      out_specs=[
          pl.BlockSpec((BT, H), lambda i: (i, 0)),
          pl.BlockSpec((BT, H), lambda i: (i, 0)),
      ],
      out_shape=[
          jax.ShapeDtypeStruct((N_PAD, H), _f32),
          jax.ShapeDtypeStruct((N_PAD, H), _f32),
      ],
  )(agg0, agg1, h0, h1, msum_parts, deg_parts)


def _tc2(f0, f1, w2, b2):
  return pl.pallas_call(
      _tc2_kernel,
      grid=(GRID_T,),
      in_specs=[
          pl.BlockSpec((BT, H), lambda i: (i, 0)),
          pl.BlockSpec((BT, H), lambda i: (i, 0)),
          pl.BlockSpec((C, C), lambda i: (0, 0)),
          pl.BlockSpec((1, C), lambda i: (0, 0)),
      ],
      out_specs=pl.BlockSpec((BT, C), lambda i: (i, 0)),
      out_shape=jax.ShapeDtypeStruct((N_PAD, C), _f32),
  )(f0, f1, w2, b2)


def kernel(x, edge_index, w1, b1, w2, b2):
  row = edge_index[0]
  col = edge_index[1]
  row_p = jnp.concatenate(
      [row, jnp.full((E_PAD - E,), N_PAD - 1, jnp.int32)])
  col_p = jnp.concatenate([col, jnp.zeros((E_PAD - E,), jnp.int32)])
  x_p = jnp.pad(x, ((0, N_PAD - N), (0, 0)))

  deg_parts = _sc_deg(row_p)
  h0, h1, dis2d = _tc1(x_p, w1, b1.reshape(1, C), deg_parts)
  dis = dis2d.reshape(N_PAD)

  f0, f1 = h0, h1
  for _ in range(K_ITERS):
    fr0, fr1, fc0, fc1, disr, disc = _sc_gather(f0, f1, row_p, col_p, dis)
    msg0, msg1, mvals = _tcN(fr0, fr1, fc0, fc1, disr, disc)
    agg0, agg1, msum_parts = _sc_scatter(msg0, msg1, row_p, mvals)
    f0, f1 = _tcC(agg0, agg1, h0, h1, msum_parts, deg_parts)

  out = _tc2(f0, f1, w2, b2.reshape(1, C))
  return out[:N]


# speedup vs baseline: 3.4543x; 1.2575x over previous
import dataclasses

import jax
import jax.numpy as jnp
from jax import lax
from jax.experimental import pallas as pl
from jax.experimental.pallas import tpu as pltpu
from jax.experimental.pallas import tpu_sc as plsc

N = 10000
E = 160000
C = 256
H = 128
NC = 2
NS = 16
NW = NC * NS
N_PAD = 10240
E_PAD = 163840
CH = 128
EA = E_PAD // NW
NCH_A = EA // CH
EB = E_PAD // NS
NCH_B = EB // CH
RPT = N_PAD // NS
MU = 0.1
P_EXP = 2.5
EPS = 1e-12
CP = 2.0 * MU / P_EXP
K_ITERS = 2
BT = 512
GRID_T = N_PAD // BT
BE = 2048
GRID_E = E_PAD // BE
CHG = 80
NCHG = EA // CHG


def _sc_params():
  cp = pltpu.CompilerParams()
  if "needs_layout_passes" in pltpu.CompilerParams.__dataclass_fields__:
    cp = dataclasses.replace(cp, needs_layout_passes=False)
  return cp


def _mesh():
  return plsc.VectorSubcoreMesh(core_axis_name="c", subcore_axis_name="s",
                                num_cores=NC, num_subcores=NS)


def _zero_ref(ref, nwords):
  z16 = jnp.zeros((16,), jnp.float32)

  @pl.loop(0, nwords // 16)
  def _(i):
    ref[pl.ds(i * 16, 16)] = z16


def _deg_body(row_hbm, deg_hbm, degb, rowb, sem):
  del sem
  wid = lax.axis_index("s") * NC + lax.axis_index("c")
  _zero_ref(degb, N_PAD)
  one16 = jnp.ones((16,), jnp.float32)

  @pl.loop(0, NCH_A)
  def _(k):
    base = wid * EA + k * CH
    pltpu.sync_copy(row_hbm.at[pl.ds(base, CH)], rowb)
    for g in range(CH // 16):
      ridx = rowb[pl.ds(g * 16, 16)]
      plsc.addupdate_scatter(degb, [ridx], one16)

  pltpu.sync_copy(degb, deg_hbm.at[wid])


def _gath_body(f_hbm, row_hbm, col_hbm, fr_hbm, fc_hbm,
               rowb0, colb0, rowb1, colb1, br0, bc0, br1, bc1,
               sg0, sg1, sw0, sw1):
  wid = lax.axis_index("s") * NC + lax.axis_index("c")
  tb = wid * EA

  def load_idx(k, rb, cb):
    pltpu.sync_copy(row_hbm.at[pl.ds(tb + k * CHG, CHG)], rb)
    pltpu.sync_copy(col_hbm.at[pl.ds(tb + k * CHG, CHG)], cb)

  def fire_g(rb, cb, br, bc, sg):
    pltpu.async_copy(f_hbm.at[rb], br, sg)
    pltpu.async_copy(f_hbm.at[cb], bc, sg)

  def drain_g(rb, cb, br, bc, sg):
    pltpu.make_async_copy(f_hbm.at[rb], br, sg).wait()
    pltpu.make_async_copy(f_hbm.at[cb], bc, sg).wait()

  def fire_w(k, br, bc, sw):
    sl = pl.ds(tb + k * CHG, CHG)
    pltpu.async_copy(br, fr_hbm.at[sl], sw)
    pltpu.async_copy(bc, fc_hbm.at[sl], sw)

  def drain_w(br, bc, sw):
    sl = pl.ds(tb, CHG)
    pltpu.make_async_copy(br, fr_hbm.at[sl], sw).wait()
    pltpu.make_async_copy(bc, fc_hbm.at[sl], sw).wait()

  load_idx(0, rowb0, colb0)
  fire_g(rowb0, colb0, br0, bc0, sg0)
  load_idx(1, rowb1, colb1)
  fire_g(rowb1, colb1, br1, bc1, sg1)

  @pl.loop(0, NCHG // 2)
  def _(j):
    k = j * 2
    drain_g(rowb0, colb0, br0, bc0, sg0)
    fire_w(k, br0, bc0, sw0)
    drain_g(rowb1, colb1, br1, bc1, sg1)
    fire_w(k + 1, br1, bc1, sw1)

    @pl.when(j < NCHG // 2 - 1)
    def _():
      load_idx(k + 2, rowb0, colb0)
      drain_w(br0, bc0, sw0)
      fire_g(rowb0, colb0, br0, bc0, sg0)
      load_idx(k + 3, rowb1, colb1)
      drain_w(br1, bc1, sw1)
      fire_g(rowb1, colb1, br1, bc1, sg1)

  drain_w(br0, bc0, sw0)
  drain_w(br1, bc1, sw1)


def _edis_body(row_hbm, col_hbm, dis_hbm, disr_hbm, disc_hbm,
               disb, rowb, colb, drb, dcb, sem):
  del sem
  wid = lax.axis_index("s") * NC + lax.axis_index("c")
  pltpu.sync_copy(dis_hbm, disb)

  @pl.loop(0, NCH_A)
  def _(k):
    base = wid * EA + k * CH
    pltpu.sync_copy(row_hbm.at[pl.ds(base, CH)], rowb)
    pltpu.sync_copy(col_hbm.at[pl.ds(base, CH)], colb)
    for g in range(CH // 16):
      sl = pl.ds(g * 16, 16)
      drb[sl] = plsc.load_gather(disb, [rowb[sl]])
      dcb[sl] = plsc.load_gather(disb, [colb[sl]])
    sl = pl.ds(base, CH)
    pltpu.sync_copy(drb, disr_hbm.at[sl])
    pltpu.sync_copy(dcb, disc_hbm.at[sl])


def _scat_body(msg0, msg1, row_hbm, m_hbm, agg0_hbm, agg1_hbm, msum_hbm,
               aggsh, msgb, rowb, mb, msumb, zb, sem):
  del sem
  cid = lax.axis_index("c")
  sid = lax.axis_index("s")
  z16 = jnp.zeros((16,), jnp.float32)
  for j in range(64):
    for l in range(H // 16):
      zb[j, pl.ds(l * 16, 16)] = z16
  _zero_ref(msumb, N_PAD)

  @pl.loop(0, RPT // 64)
  def _(j):
    pltpu.sync_copy(zb, aggsh.at[pl.ds(sid * RPT + j * 64, 64)])

  plsc.subcore_barrier()

  @pl.loop(0, NCH_B)
  def _(k):
    base = sid * EB + k * CH
    pltpu.sync_copy(row_hbm.at[pl.ds(base, CH)], rowb)

    @pl.when(cid == 0)
    def _():
      pltpu.sync_copy(m_hbm.at[pl.ds(base, CH)], mb)
      for g in range(CH // 16):
        sl = pl.ds(g * 16, 16)
        plsc.addupdate_scatter(msumb, [rowb[sl]], mb[sl])
      pltpu.sync_copy(msg0.at[pl.ds(base, CH)], msgb)

    @pl.when(cid == 1)
    def _():
      pltpu.sync_copy(msg1.at[pl.ds(base, CH)], msgb)

    pltpu.sync_copy(msgb, aggsh.at[rowb], add=True)

  plsc.subcore_barrier()

  @pl.loop(0, RPT // 64)
  def _(j):
    r = sid * RPT + j * 64

    @pl.when(cid == 0)
    def _():
      pltpu.sync_copy(aggsh.at[pl.ds(r, 64)], agg0_hbm.at[pl.ds(r, 64)])

    @pl.when(cid == 1)
    def _():
      pltpu.sync_copy(aggsh.at[pl.ds(r, 64)], agg1_hbm.at[pl.ds(r, 64)])

  @pl.when(cid == 0)
  def _():
    pltpu.sync_copy(msumb, msum_hbm.at[sid])


_DN_ROW = (((1,), (0,)), ((), ()))
_DN_COL = (((0,), (0,)), ((), ()))


def _tc1_kernel(x_ref, w1_ref, b1_ref, degp_ref, h_ref, dis_ref):
  h = lax.dot_general(x_ref[...], w1_ref[...], _DN_ROW,
                      preferred_element_type=jnp.float32,
                      precision=lax.Precision.HIGHEST)
  h_ref[...] = jnp.maximum(h + b1_ref[...], 0.0)
  deg = jnp.sum(degp_ref[...], axis=0, keepdims=True)
  dis_ref[...] = jnp.where(deg > 0, lax.rsqrt(jnp.maximum(deg, EPS)), 0.0)


def _tcN_kernel(fr_ref, fc_ref, dr_ref, dc_ref, msg0_ref, msg1_ref, m_ref):
  a = dr_ref[...]
  b = dc_ref[...]
  fc = fc_ref[...]
  g = b * fc - a * fr_ref[...]
  u = jnp.sum(g * g, axis=1, keepdims=True) + EPS
  m = jnp.sqrt(jnp.sqrt(u))
  c = m * a * b
  msg0_ref[...] = c * fc[:, :H]
  msg1_ref[...] = c * fc[:, H:]
  m_ref[...] = m


def _tcC_kernel(agg0_ref, agg1_ref, h_ref, msump_ref, degp_ref, f_ref):
  ones_s = jnp.ones((NS, 1), jnp.float32)
  ones_w = jnp.ones((NW, 1), jnp.float32)
  msum = lax.dot_general(msump_ref[...], ones_s, _DN_COL,
                         preferred_element_type=jnp.float32,
                         precision=lax.Precision.HIGHEST)
  deg = lax.dot_general(degp_ref[...], ones_w, _DN_COL,
                        preferred_element_type=jnp.float32,
                        precision=lax.Precision.HIGHEST)
  alpha = 1.0 / (msum / jnp.maximum(deg, EPS) + CP)
  beta = CP * alpha
  agg = jnp.concatenate([agg0_ref[...], agg1_ref[...]], axis=1)
  f_ref[...] = alpha * agg + beta * h_ref[...]


def _tc2_kernel(f_ref, w2_ref, b2_ref, o_ref):
  z = lax.dot_general(f_ref[...], w2_ref[...], _DN_ROW,
                      preferred_element_type=jnp.float32,
                      precision=lax.Precision.HIGHEST)
  z += b2_ref[...]
  m = jnp.max(z, axis=1, keepdims=True)
  o_ref[...] = z - m - jnp.log(jnp.sum(jnp.exp(z - m), axis=1, keepdims=True))


_f32 = jnp.float32


def _sc_deg(row_p):
  fn = pl.kernel(
      _deg_body,
      out_type=jax.ShapeDtypeStruct((NW, N_PAD), _f32),
      mesh=_mesh(),
      compiler_params=_sc_params(),
      scratch_types=[
          pltpu.VMEM((N_PAD,), _f32),
          pltpu.VMEM((CH,), jnp.int32),
          pltpu.SemaphoreType.DMA,
      ],
  )
  return fn(row_p)


def _sc_edis(row_p, col_p, dis):
  fn = pl.kernel(
      _edis_body,
      out_type=(jax.ShapeDtypeStruct((E_PAD,), _f32),
                jax.ShapeDtypeStruct((E_PAD,), _f32)),
      mesh=_mesh(),
      compiler_params=_sc_params(),
      scratch_types=[
          pltpu.VMEM((N_PAD,), _f32),
          pltpu.VMEM((CH,), jnp.int32),
          pltpu.VMEM((CH,), jnp.int32),
          pltpu.VMEM((CH,), _f32),
          pltpu.VMEM((CH,), _f32),
          pltpu.SemaphoreType.DMA,
      ],
  )
  return fn(row_p, col_p, dis)


def _sc_gather(f, row_p, col_p):
  fn = pl.kernel(
      _gath_body,
      out_type=(jax.ShapeDtypeStruct((E_PAD, C), _f32),
                jax.ShapeDtypeStruct((E_PAD, C), _f32)),
      mesh=_mesh(),
      compiler_params=_sc_params(),
      scratch_types=[
          pltpu.VMEM((CHG,), jnp.int32),
          pltpu.VMEM((CHG,), jnp.int32),
          pltpu.VMEM((CHG,), jnp.int32),
          pltpu.VMEM((CHG,), jnp.int32),
          pltpu.VMEM((CHG, C), _f32),
          pltpu.VMEM((CHG, C), _f32),
          pltpu.VMEM((CHG, C), _f32),
          pltpu.VMEM((CHG, C), _f32),
          pltpu.SemaphoreType.DMA,
          pltpu.SemaphoreType.DMA,
          pltpu.SemaphoreType.DMA,
          pltpu.SemaphoreType.DMA,
      ],
  )
  return fn(f, row_p, col_p)


def _sc_scatter(msg0, msg1, row_p, mvals):
  fn = pl.kernel(
      _scat_body,
      out_type=(jax.ShapeDtypeStruct((N_PAD, H), _f32),
                jax.ShapeDtypeStruct((N_PAD, H), _f32),
                jax.ShapeDtypeStruct((NS, N_PAD), _f32)),
      mesh=_mesh(),
      compiler_params=_sc_params(),
      scratch_types=[
          pltpu.VMEM_SHARED((N_PAD, H), _f32),
          pltpu.VMEM((CH, H), _f32),
          pltpu.VMEM((CH,), jnp.int32),
          pltpu.VMEM((CH,), _f32),
          pltpu.VMEM((N_PAD,), _f32),
          pltpu.VMEM((64, H), _f32),
          pltpu.SemaphoreType.DMA,
      ],
  )
  return fn(msg0, msg1, row_p, mvals)


def _tc1(x_p, w1, b1, deg_parts):
  return pl.pallas_call(
      _tc1_kernel,
      grid=(GRID_T,),
      in_specs=[
          pl.BlockSpec((BT, C), lambda i: (i, 0)),
          pl.BlockSpec((C, C), lambda i: (0, 0)),
          pl.BlockSpec((1, C), lambda i: (0, 0)),
          pl.BlockSpec((NW, BT), lambda i: (0, i)),
      ],
      out_specs=[
          pl.BlockSpec((BT, C), lambda i: (i, 0)),
          pl.BlockSpec((1, BT), lambda i: (0, i)),
      ],
      out_shape=[
          jax.ShapeDtypeStruct((N_PAD, C), _f32),
          jax.ShapeDtypeStruct((1, N_PAD), _f32),
      ],
  )(x_p, w1, b1, deg_parts)


def _tcN(fr, fc, disr, disc):
  dr = disr.reshape(E_PAD, 1)
  dc = disc.reshape(E_PAD, 1)
  msg0, msg1, m2d = pl.pallas_call(
      _tcN_kernel,
      grid=(GRID_E,),
      in_specs=[
          pl.BlockSpec((BE, C), lambda i: (i, 0)),
          pl.BlockSpec((BE, C), lambda i: (i, 0)),
          pl.BlockSpec((BE, 1), lambda i: (i, 0)),
          pl.BlockSpec((BE, 1), lambda i: (i, 0)),
      ],
      out_specs=[
          pl.BlockSpec((BE, H), lambda i: (i, 0)),
          pl.BlockSpec((BE, H), lambda i: (i, 0)),
          pl.BlockSpec((BE, 1), lambda i: (i, 0)),
      ],
      out_shape=[
          jax.ShapeDtypeStruct((E_PAD, H), _f32),
          jax.ShapeDtypeStruct((E_PAD, H), _f32),
          jax.ShapeDtypeStruct((E_PAD, 1), _f32),
      ],
  )(fr, fc, dr, dc)
  return msg0, msg1, m2d.reshape(E_PAD)


def _tcC(agg0, agg1, h, msum_parts, deg_parts):
  return pl.pallas_call(
      _tcC_kernel,
      grid=(GRID_T,),
      in_specs=[
          pl.BlockSpec((BT, H), lambda i: (i, 0)),
          pl.BlockSpec((BT, H), lambda i: (i, 0)),
          pl.BlockSpec((BT, C), lambda i: (i, 0)),
          pl.BlockSpec((NS, BT), lambda i: (0, i)),
          pl.BlockSpec((NW, BT), lambda i: (0, i)),
      ],
      out_specs=pl.BlockSpec((BT, C), lambda i: (i, 0)),
      out_shape=jax.ShapeDtypeStruct((N_PAD, C), _f32),
  )(agg0, agg1, h, msum_parts, deg_parts)


def _tc2(f, w2, b2):
  return pl.pallas_call(
      _tc2_kernel,
      grid=(GRID_T,),
      in_specs=[
          pl.BlockSpec((BT, C), lambda i: (i, 0)),
          pl.BlockSpec((C, C), lambda i: (0, 0)),
          pl.BlockSpec((1, C), lambda i: (0, 0)),
      ],
      out_specs=pl.BlockSpec((BT, C), lambda i: (i, 0)),
      out_shape=jax.ShapeDtypeStruct((N_PAD, C), _f32),
  )(f, w2, b2)


def kernel(x, edge_index, w1, b1, w2, b2):
  row = edge_index[0]
  col = edge_index[1]
  row_p = jnp.concatenate(
      [row, jnp.full((E_PAD - E,), N_PAD - 1, jnp.int32)])
  col_p = jnp.concatenate([col, jnp.zeros((E_PAD - E,), jnp.int32)])
  x_p = jnp.pad(x, ((0, N_PAD - N), (0, 0)))

  deg_parts = _sc_deg(row_p)
  h, dis2d = _tc1(x_p, w1, b1.reshape(1, C), deg_parts)
  dis = dis2d.reshape(N_PAD)
  disr, disc = _sc_edis(row_p, col_p, dis)

  f = h
  for _ in range(K_ITERS):
    fr, fc = _sc_gather(f, row_p, col_p)
    msg0, msg1, mvals = _tcN(fr, fc, disr, disc)
    agg0, agg1, msum_parts = _sc_scatter(msg0, msg1, row_p, mvals)
    f = _tcC(agg0, agg1, h, msum_parts, deg_parts)

  out = _tc2(f, w2, b2.reshape(1, C))
  return out[:N]
